# Initial kernel scaffold; baseline (speedup 1.0000x reference)
import jax
import jax.numpy as jnp
from jax.experimental import pallas as pl


def kernel(x, W1, b1, gamma1, beta1, W2, b2, edge_index):
    raise NotImplementedError("write your pallas kernel here")


# speedup vs baseline: 13.3614x; 13.3614x over previous
import functools

import jax
import jax.numpy as jnp
from jax import lax
from jax.experimental import pallas as pl
from jax.experimental.pallas import tpu as pltpu
from jax.experimental.pallas import tpu_sc as plsc

N = 10000
E = 320000
D_IN = 128
D_H = 128
D_OUT = 40

NC = 2
NS = 16
NW = NC * NS
CH = 128
K = 80
E_PAD = NW * K * CH
N_PAD = 10112
RPT = N_PAD // NS


def _mesh():
    return plsc.VectorSubcoreMesh(
        core_axis_name="c", subcore_axis_name="s", num_cores=NC, num_subcores=NS
    )


@functools.cache
def _make_degree():
    return functools.partial(
        pl.kernel,
        out_type=jax.ShapeDtypeStruct((NC * N_PAD,), jnp.float32),
        mesh=_mesh(),
        scratch_types=[
            pltpu.VMEM((K, CH), jnp.int32),
            pltpu.VMEM((CH,), jnp.float32),
            pltpu.VMEM((RPT,), jnp.float32),
            pltpu.VMEM_SHARED((N_PAD,), jnp.float32),
            pltpu.SemaphoreType.DMA,
        ],
    )(_sc_degree_body)


def _sc_degree_body(dst_hbm, out_hbm, didx, ones_v, zbuf, acc, sem):
    c = lax.axis_index("c")
    s = lax.axis_index("s")
    wid = c * NS + s

    pltpu.sync_copy(dst_hbm.at[wid], didx)

    for i in range(CH // 16):
        ones_v[pl.ds(i * 16, 16)] = jnp.ones((16,), jnp.float32)

    def zfill(i, _):
        zbuf[pl.ds(i * 16, 16)] = jnp.zeros((16,), jnp.float32)
        return 0

    lax.fori_loop(0, RPT // 16, zfill, 0)
    zbuf[pl.ds(RPT - 16, 16)] = jnp.zeros((16,), jnp.float32)

    pltpu.sync_copy(zbuf, acc.at[pl.ds(s * RPT, RPT)])
    plsc.subcore_barrier()

    def body(jj, _):
        descs = [
            pltpu.async_copy(ones_v, acc.at[didx.at[jj * 8 + b]], sem, add=True)
            for b in range(8)
        ]
        for d in descs:
            d.wait()
        return 0

    lax.fori_loop(0, K // 8, body, 0)
    plsc.subcore_barrier()
    pltpu.sync_copy(acc.at[pl.ds(s * RPT, RPT)], zbuf)
    pltpu.sync_copy(zbuf, out_hbm.at[pl.ds(c * N_PAD + s * RPT, RPT)])


G = 16
NG = K // G


@functools.cache
def _make_scatter(D):
    params = (None if D % 128 == 0
              else pltpu.CompilerParams(use_tc_tiling_on_sc=False))

    @functools.partial(
        pl.kernel,
        out_type=jax.ShapeDtypeStruct((NC, N_PAD, D), jnp.float32),
        compiler_params=params,
        mesh=_mesh(),
        scratch_types=[
            pltpu.VMEM((2, G, CH), jnp.int32),
            pltpu.VMEM((2, G, CH), jnp.int32),
            pltpu.VMEM((2, CH, D), jnp.float32),
            pltpu.VMEM((8, D), jnp.float32),
            pltpu.VMEM_SHARED((N_PAD, D), jnp.float32),
            pltpu.SemaphoreType.DMA,
            pltpu.SemaphoreType.DMA,
            pltpu.SemaphoreType.DMA,
            pltpu.SemaphoreType.DMA,
        ],
    )
    def scatter(src_hbm, dst_hbm, h_hbm, out_hbm, sidx, didx, rows, zrows,
                acc, sem_z, sem_i, sem_g0, sem_g1):
        c = lax.axis_index("c")
        s = lax.axis_index("s")
        wid = c * NS + s

        pltpu.sync_copy(src_hbm.at[wid, pl.ds(0, G)], sidx.at[0])
        pltpu.sync_copy(dst_hbm.at[wid, pl.ds(0, G)], didx.at[0])
        pltpu.async_copy(src_hbm.at[wid, pl.ds(G, G)], sidx.at[1], sem_i)
        pltpu.async_copy(dst_hbm.at[wid, pl.ds(G, G)], didx.at[1], sem_i)

        col_offs = list(range(0, D - 15, 16))
        if D % 16:
            col_offs.append(D - 16)
        for r in range(8):
            for o in col_offs:
                zrows[r, pl.ds(o, 16)] = jnp.zeros((16,), jnp.float32)

        def zbody(r, _):
            pltpu.async_copy(zrows, acc.at[pl.ds(s * RPT + r * 8, 8)], sem_z)
            return 0

        lax.fori_loop(0, RPT // 8, zbody, 0)

        def zdrain(r, _):
            pltpu.make_async_copy(zrows, acc.at[pl.ds(s * RPT + r * 8, 8)], sem_z).wait()
            return 0

        lax.fori_loop(0, RPT // 8, zdrain, 0)
        plsc.subcore_barrier()

        sems = (sem_g0, sem_g1)
        for g in range(NG):
            bb = g & 1
            if g > 0:
                pltpu.make_async_copy(
                    src_hbm.at[wid, pl.ds(g * G, G)], sidx.at[bb], sem_i).wait()
                pltpu.make_async_copy(
                    dst_hbm.at[wid, pl.ds(g * G, G)], didx.at[bb], sem_i).wait()

            for b in range(2):
                pltpu.async_copy(h_hbm.at[sidx.at[bb, b]], rows.at[b], sems[b])

            def body(jj, _, bb=bb):
                for b in range(2):
                    j = jj * 2 + b
                    pltpu.make_async_copy(
                        h_hbm.at[sidx.at[bb, j]], rows.at[b], sems[b]).wait()
                    pltpu.sync_copy(rows.at[b], acc.at[didx.at[bb, j]], add=True)
                    pltpu.async_copy(
                        h_hbm.at[sidx.at[bb, j + 2]], rows.at[b], sems[b])
                return 0

            lax.fori_loop(0, G // 2 - 1, body, 0)
            for b in range(2):
                j = G - 2 + b
                pltpu.make_async_copy(
                    h_hbm.at[sidx.at[bb, j]], rows.at[b], sems[b]).wait()
                pltpu.sync_copy(rows.at[b], acc.at[didx.at[bb, j]], add=True)

            if g + 2 < NG:
                pltpu.async_copy(
                    src_hbm.at[wid, pl.ds((g + 2) * G, G)], sidx.at[bb], sem_i)
                pltpu.async_copy(
                    dst_hbm.at[wid, pl.ds((g + 2) * G, G)], didx.at[bb], sem_i)

        plsc.subcore_barrier()
        pltpu.sync_copy(acc.at[pl.ds(s * RPT, RPT)],
                        out_hbm.at[c, pl.ds(s * RPT, RPT)])

    return scatter


def _tc1_body(x_ref, w_ref, degp_ref, hp_ref, s_ref):
    deg = degp_ref[0] + degp_ref[1] + 1.0
    s = lax.rsqrt(jnp.maximum(deg, 1.0))
    s_ref[...] = s
    h = jnp.dot(x_ref[...], w_ref[...], preferred_element_type=jnp.float32)
    hp_ref[:N] = h * s[:N]
    hp_ref[N:] = jnp.zeros((N_PAD - N, D_H), jnp.float32)


def _tc1(x, W1, degp):
    return pl.pallas_call(
        _tc1_body,
        out_shape=(
            jax.ShapeDtypeStruct((N_PAD, D_H), jnp.float32),
            jax.ShapeDtypeStruct((N_PAD, 1), jnp.float32),
        ),
    )(x, W1, degp)


def _tc2_body(aggp_ref, hp_ref, s_ref, b1_ref, g1_ref, be1_ref, w2_ref, h2p_ref):
    sN = s_ref[:N]
    a = sN * (aggp_ref[0][:N] + aggp_ref[1][:N] + hp_ref[:N]) + b1_ref[...]
    mean = jnp.mean(a, axis=0, keepdims=True)
    cen = a - mean
    var = jnp.mean(cen * cen, axis=0, keepdims=True)
    h1 = jnp.maximum(g1_ref[...] * cen * lax.rsqrt(var + 1e-5) + be1_ref[...], 0.0)
    h2 = jnp.dot(h1, w2_ref[...], preferred_element_type=jnp.float32)
    h2p_ref[:N] = h2 * sN
    h2p_ref[N:] = jnp.zeros((N_PAD - N, D_OUT), jnp.float32)


def _tc2(aggp, hp, s, b1, g1, be1, W2):
    return pl.pallas_call(
        _tc2_body,
        out_shape=jax.ShapeDtypeStruct((N_PAD, D_OUT), jnp.float32),
    )(aggp, hp, s, b1, g1, be1, W2)


def _tc3_body(agg2p_ref, h2p_ref, s_ref, b2_ref, out_ref):
    o = s_ref[:N] * (agg2p_ref[0][:N] + agg2p_ref[1][:N] + h2p_ref[:N]) + b2_ref[...]
    m = jnp.max(o, axis=1, keepdims=True)
    lse = jnp.log(jnp.sum(jnp.exp(o - m), axis=1, keepdims=True))
    out_ref[...] = o - m - lse


def _tc3(agg2p, h2p, s, b2):
    return pl.pallas_call(
        _tc3_body,
        out_shape=jax.ShapeDtypeStruct((N, D_OUT), jnp.float32),
    )(agg2p, h2p, s, b2)


def kernel(x, W1, b1, gamma1, beta1, W2, b2, edge_index):
    pad = jnp.full((E_PAD - E,), N, jnp.int32)
    srcp = jnp.concatenate([edge_index[0], pad]).reshape(NW, K, CH)
    dstp = jnp.concatenate([edge_index[1], pad]).reshape(NW, K, CH)

    degp = _make_degree()(dstp).reshape(NC, N_PAD, 1)
    hp, s = _tc1(x, W1, degp)
    aggp = _make_scatter(D_H)(srcp, dstp, hp)
    h2p = _tc2(aggp, hp, s, b1.reshape(1, D_H), gamma1.reshape(1, D_H),
               beta1.reshape(1, D_H), W2)
    agg2p = _make_scatter(D_OUT)(srcp, dstp, h2p)
    return _tc3(agg2p, h2p, s, b2.reshape(1, D_OUT))


# speedup vs baseline: 17.8131x; 1.3332x over previous
import functools

import jax
import jax.numpy as jnp
from jax import lax
from jax.experimental import pallas as pl
from jax.experimental.pallas import tpu as pltpu
from jax.experimental.pallas import tpu_sc as plsc

N = 10000
E = 320000
D_IN = 128
D_H = 128
D_OUT = 40
D2H = D_H // 2
D_OP = 48
D2O = D_OP // 2

NC = 2
NS = 16
CH = 128
K2 = 160
E_PAD = NS * K2 * CH
N_PAD = 10112
RPT = N_PAD // NS
NBUF = 4

_UNTILED = pltpu.CompilerParams(use_tc_tiling_on_sc=False)


def _mesh():
    return plsc.VectorSubcoreMesh(
        core_axis_name="c", subcore_axis_name="s", num_cores=NC, num_subcores=NS
    )


KD = K2 // NC


@functools.cache
def _make_degree():
    return functools.partial(
        pl.kernel,
        out_type=jax.ShapeDtypeStruct((NC * N_PAD,), jnp.float32),
        mesh=_mesh(),
        scratch_types=[
            pltpu.VMEM((KD, CH), jnp.int32),
            pltpu.VMEM((CH,), jnp.float32),
            pltpu.VMEM((RPT,), jnp.float32),
            pltpu.VMEM_SHARED((N_PAD,), jnp.float32),
            pltpu.SemaphoreType.DMA,
        ],
    )(_sc_degree_body)


def _sc_degree_body(dst_hbm, out_hbm, didx, ones_v, zbuf, acc, sem):
    c = lax.axis_index("c")
    s = lax.axis_index("s")
    wid = c * NS + s

    pltpu.sync_copy(dst_hbm.at[wid], didx)

    for i in range(CH // 16):
        ones_v[pl.ds(i * 16, 16)] = jnp.ones((16,), jnp.float32)

    def zfill(i, _):
        zbuf[pl.ds(i * 16, 16)] = jnp.zeros((16,), jnp.float32)
        return 0

    lax.fori_loop(0, RPT // 16, zfill, 0)
    zbuf[pl.ds(RPT - 16, 16)] = jnp.zeros((16,), jnp.float32)

    pltpu.sync_copy(zbuf, acc.at[pl.ds(s * RPT, RPT)])
    plsc.subcore_barrier()

    def body(jj, _):
        descs = [
            pltpu.async_copy(ones_v, acc.at[didx.at[jj * 8 + b]], sem, add=True)
            for b in range(8)
        ]
        for d in descs:
            d.wait()
        return 0

    lax.fori_loop(0, KD // 8, body, 0)
    plsc.subcore_barrier()
    pltpu.sync_copy(acc.at[pl.ds(s * RPT, RPT)], zbuf)
    pltpu.sync_copy(zbuf, out_hbm.at[pl.ds(c * N_PAD + s * RPT, RPT)])


@functools.cache
def _make_scatter(D2):
    @functools.partial(
        pl.kernel,
        out_type=jax.ShapeDtypeStruct((NC * N_PAD, D2), jnp.float32),
        compiler_params=_UNTILED,
        mesh=_mesh(),
        scratch_types=[
            pltpu.VMEM((K2, CH), jnp.int32),
            pltpu.VMEM((K2, CH), jnp.int32),
            pltpu.VMEM((NBUF, CH, D2), jnp.float32),
            pltpu.VMEM_SHARED((N_PAD, D2), jnp.float32),
            pltpu.SemaphoreType.DMA,
            pltpu.SemaphoreType.DMA,
            pltpu.SemaphoreType.DMA,
            pltpu.SemaphoreType.DMA,
            pltpu.SemaphoreType.DMA,
            pltpu.SemaphoreType.DMA,
            pltpu.SemaphoreType.DMA,
            pltpu.SemaphoreType.DMA,
            pltpu.SemaphoreType.DMA,
        ],
    )
    def scatter(src_hbm, dst_hbm, h_hbm, out_hbm, sidx, didx, rows, acc,
                sem_z, sg0, sg1, sg2, sg3, ss0, ss1, ss2, ss3):
        c = lax.axis_index("c")
        s = lax.axis_index("s")
        sg = (sg0, sg1, sg2, sg3)
        ss = (ss0, ss1, ss2, ss3)

        pltpu.sync_copy(src_hbm.at[c, s], sidx)
        pltpu.sync_copy(dst_hbm.at[s], didx)

        col_offs = list(range(0, D2 - 15, 16))
        if D2 % 16:
            col_offs.append(D2 - 16)
        for r in range(8):
            for o in col_offs:
                rows[0, r, pl.ds(o, 16)] = jnp.zeros((16,), jnp.float32)

        def zbody(r, _):
            pltpu.async_copy(rows.at[0, pl.ds(0, 8)],
                             acc.at[pl.ds(s * RPT + r * 8, 8)], sem_z)
            return 0

        lax.fori_loop(0, RPT // 8, zbody, 0)

        def zdrain(r, _):
            pltpu.make_async_copy(rows.at[0, pl.ds(0, 8)],
                                  acc.at[pl.ds(s * RPT + r * 8, 8)], sem_z).wait()
            return 0

        lax.fori_loop(0, RPT // 8, zdrain, 0)
        plsc.subcore_barrier()

        def gath(j, b):
            pltpu.async_copy(h_hbm.at[sidx.at[j]], rows.at[b], sg[b])

        def gwait(j, b):
            pltpu.make_async_copy(h_hbm.at[sidx.at[j]], rows.at[b], sg[b]).wait()

        def scat(j, b):
            pltpu.async_copy(rows.at[b], acc.at[didx.at[j]], ss[b], add=True)

        def swaitf(j, b):
            pltpu.make_async_copy(rows.at[b], acc.at[didx.at[j]], ss[b]).wait()

        for b in range(NBUF - 1):
            gath(b, b)

        def step(jj, b):
            j = jj * NBUF + b
            gwait(j, b)
            scat(j, b)
            mb = (b + NBUF - 1) % NBUF
            swaitf(j - 1, mb)
            gath(j + NBUF - 1, mb)

        gwait(0, 0)
        scat(0, 0)
        gath(NBUF - 1, NBUF - 1)
        for b in range(1, NBUF):
            step(0, b)

        def body(jj, _):
            for b in range(NBUF):
                step(jj, b)
            return 0

        lax.fori_loop(1, K2 // NBUF - 1, body, 0)

        jl = K2 // NBUF - 1
        step(jl, 0)
        for b in range(1, NBUF):
            j = jl * NBUF + b
            gwait(j, b)
            scat(j, b)
            swaitf(j - 1, (b + NBUF - 1) % NBUF)
        swaitf(K2 - 1, NBUF - 1)

        plsc.subcore_barrier()
        pltpu.sync_copy(acc.at[pl.ds(s * RPT, RPT)],
                        out_hbm.at[pl.ds(c * N_PAD + s * RPT, RPT)])

    return scatter


def _tc1_body(x_ref, w_ref, degp_ref, hp_ref, s_ref):
    deg = degp_ref[0] + degp_ref[1] + 1.0
    sv = lax.rsqrt(jnp.maximum(deg, 1.0))
    s_ref[...] = sv
    h = jnp.dot(x_ref[...], w_ref[...], preferred_element_type=jnp.float32)
    hs = h * sv[:N]
    z = jnp.zeros((N_PAD - N, D2H), jnp.float32)
    hp_ref[:N] = hs[:, :D2H]
    hp_ref[N:N_PAD] = z
    hp_ref[N_PAD:N_PAD + N] = hs[:, D2H:]
    hp_ref[N_PAD + N:] = z


def _tc1(x, W1, degp):
    return pl.pallas_call(
        _tc1_body,
        out_shape=(
            jax.ShapeDtypeStruct((NC * N_PAD, D2H), jnp.float32),
            jax.ShapeDtypeStruct((N_PAD, 1), jnp.float32),
        ),
    )(x, W1, degp)


def _tc2_body(aggp_ref, hp_ref, s_ref, b1_ref, g1_ref, be1_ref, w2_ref, h2p_ref):
    sN = s_ref[:N]
    left = aggp_ref[:N] + hp_ref[:N]
    right = aggp_ref[N_PAD:N_PAD + N] + hp_ref[N_PAD:N_PAD + N]
    a = sN * jnp.concatenate([left, right], axis=1) + b1_ref[...]
    mean = jnp.mean(a, axis=0, keepdims=True)
    cen = a - mean
    var = jnp.mean(cen * cen, axis=0, keepdims=True)
    h1 = jnp.maximum(g1_ref[...] * cen * lax.rsqrt(var + 1e-5) + be1_ref[...], 0.0)
    h2 = jnp.dot(h1, w2_ref[...], preferred_element_type=jnp.float32)
    h2s = h2 * sN
    z = jnp.zeros((N_PAD - N, D2O), jnp.float32)
    h2p_ref[:N] = h2s[:, :D2O]
    h2p_ref[N:N_PAD] = z
    h2p_ref[N_PAD:N_PAD + N] = h2s[:, D2O:]
    h2p_ref[N_PAD + N:] = z


def _tc2(aggp, hp, s, b1, g1, be1, W2):
    return pl.pallas_call(
        _tc2_body,
        out_shape=jax.ShapeDtypeStruct((NC * N_PAD, D2O), jnp.float32),
    )(aggp, hp, s, b1, g1, be1, W2)


def _tc3_body(agg2p_ref, h2p_ref, s_ref, b2_ref, out_ref):
    left = agg2p_ref[:N] + h2p_ref[:N]
    right = agg2p_ref[N_PAD:N_PAD + N] + h2p_ref[N_PAD:N_PAD + N]
    o = (s_ref[:N] * jnp.concatenate([left, right], axis=1))[:, :D_OUT] + b2_ref[...]
    m = jnp.max(o, axis=1, keepdims=True)
    lse = jnp.log(jnp.sum(jnp.exp(o - m), axis=1, keepdims=True))
    out_ref[...] = o - m - lse


def _tc3(agg2p, h2p, s, b2):
    return pl.pallas_call(
        _tc3_body,
        out_shape=jax.ShapeDtypeStruct((N, D_OUT), jnp.float32),
    )(agg2p, h2p, s, b2)


def kernel(x, W1, b1, gamma1, beta1, W2, b2, edge_index):
    pad = jnp.full((E_PAD - E,), N, jnp.int32)
    src = jnp.concatenate([edge_index[0], pad])
    dst = jnp.concatenate([edge_index[1], pad])
    dstp = dst.reshape(NS, K2, CH)
    dstp_deg = dst.reshape(NC * NS, KD, CH)
    srcp = jnp.stack([src, src + N_PAD]).reshape(NC, NS, K2, CH)

    degp = _make_degree()(dstp_deg).reshape(NC, N_PAD, 1)
    hp, s = _tc1(x, W1, degp)
    aggp = _make_scatter(D2H)(srcp, dstp, hp)
    W2p = jnp.pad(W2, ((0, 0), (0, D_OP - D_OUT)))
    h2p = _tc2(aggp, hp, s, b1.reshape(1, D_H), gamma1.reshape(1, D_H),
               beta1.reshape(1, D_H), W2p)
    agg2p = _make_scatter(D2O)(srcp, dstp, h2p)
    return _tc3(agg2p, h2p, s, b2.reshape(1, D_OUT))
